# initial kernel scaffold (unmeasured)
import jax
import jax.numpy as jnp
from jax import lax
from jax.experimental import pallas as pl
from jax.experimental.pallas import tpu as pltpu

N_DEV = 32


def kernel(x, w_mat):
    m_per, k = x.shape
    _, n = w_mat.shape
    n_per = n // N_DEV

    def body(x_ref, w_ref, out_ref, slots, send_sems, recv_sems):
        my = lax.axis_index("i")

        y = jnp.dot(x_ref[:, :], w_ref[:, :],
                    preferred_element_type=jnp.float32)
        y16 = y.astype(jnp.bfloat16)
        slots[:, :, :] = y16.reshape(m_per, N_DEV, n_per).transpose(1, 0, 2)

        out_ref[pl.ds(my * m_per, m_per), :] = slots[my]

        sends = []
        for d in range(1, N_DEV):
            j = lax.rem(my + d, N_DEV)
            rdma = pltpu.make_async_remote_copy(
                src_ref=slots.at[j],
                dst_ref=out_ref.at[pl.ds(my * m_per, m_per), :],
                send_sem=send_sems.at[j],
                recv_sem=recv_sems.at[my],
                device_id=(j,),
                device_id_type=pl.DeviceIdType.MESH,
            )
            rdma.start()
            sends.append(rdma)

        for d in range(1, N_DEV):
            i = lax.rem(my - d + N_DEV, N_DEV)
            recv = pltpu.make_async_remote_copy(
                src_ref=slots.at[i],
                dst_ref=out_ref.at[pl.ds(i * m_per, m_per), :],
                send_sem=send_sems.at[i],
                recv_sem=recv_sems.at[i],
                device_id=(i,),
                device_id_type=pl.DeviceIdType.MESH,
            )
            recv.wait_recv()

        for rdma in sends:
            rdma.wait_send()

    return pl.pallas_call(
        body,
        out_shape=jax.ShapeDtypeStruct((m_per * N_DEV, n_per), jnp.bfloat16),
        in_specs=[
            pl.BlockSpec(memory_space=pltpu.VMEM),
            pl.BlockSpec(memory_space=pltpu.VMEM),
        ],
        out_specs=pl.BlockSpec(memory_space=pltpu.VMEM),
        scratch_shapes=[
            pltpu.VMEM((N_DEV, m_per, n_per), jnp.bfloat16),
            pltpu.SemaphoreType.DMA((N_DEV,)),
            pltpu.SemaphoreType.DMA((N_DEV,)),
        ],
    )(x, w_mat)


# baseline (device time: 42941 ns/iter reference)
import jax
import jax.numpy as jnp
from jax import lax
from jax.experimental import pallas as pl
from jax.experimental.pallas import tpu as pltpu

N_DEV = 32


def kernel(x, w_mat):
    m_per, k = x.shape
    _, n = w_mat.shape
    n_per = n // N_DEV

    def body(x_ref, w_ref, out_ref, slots, send_sems, recv_sems):
        my = lax.axis_index("i")

        y = jnp.dot(x_ref[:, :].astype(jnp.bfloat16),
                    w_ref[:, :].astype(jnp.bfloat16),
                    preferred_element_type=jnp.float32)
        y16 = y.astype(jnp.bfloat16)
        slots[:, :, :] = y16.reshape(m_per, N_DEV, n_per).transpose(1, 0, 2)

        out_ref[pl.ds(my * m_per, m_per), :] = slots[my]

        sends = []
        for d in range(1, N_DEV):
            j = lax.rem(my + d, N_DEV)
            rdma = pltpu.make_async_remote_copy(
                src_ref=slots.at[j],
                dst_ref=out_ref.at[pl.ds(my * m_per, m_per), :],
                send_sem=send_sems.at[j],
                recv_sem=recv_sems.at[my],
                device_id=(j,),
                device_id_type=pl.DeviceIdType.MESH,
            )
            rdma.start()
            sends.append(rdma)

        for d in range(1, N_DEV):
            i = lax.rem(my - d + N_DEV, N_DEV)
            recv = pltpu.make_async_remote_copy(
                src_ref=slots.at[i],
                dst_ref=out_ref.at[pl.ds(i * m_per, m_per), :],
                send_sem=send_sems.at[i],
                recv_sem=recv_sems.at[i],
                device_id=(i,),
                device_id_type=pl.DeviceIdType.MESH,
            )
            recv.wait_recv()

        for rdma in sends:
            rdma.wait_send()

    return pl.pallas_call(
        body,
        out_shape=jax.ShapeDtypeStruct((m_per * N_DEV, n_per), jnp.bfloat16),
        in_specs=[
            pl.BlockSpec(memory_space=pltpu.VMEM),
            pl.BlockSpec(memory_space=pltpu.VMEM),
        ],
        out_specs=pl.BlockSpec(memory_space=pltpu.VMEM),
        scratch_shapes=[
            pltpu.VMEM((N_DEV, m_per, n_per), jnp.bfloat16),
            pltpu.SemaphoreType.DMA((N_DEV,)),
            pltpu.SemaphoreType.DMA((N_DEV,)),
        ],
        compiler_params=pltpu.CompilerParams(
            vmem_limit_bytes=100 * 1024 * 1024,
        ),
    )(x, w_mat)
